# baseline (speedup 1.0000x reference)
import functools

import jax
import jax.numpy as jnp
from jax import lax
from jax.experimental import pallas as pl
from jax.experimental.pallas import tpu as pltpu
from jax.experimental.pallas import tpu_sc as plsc

B = 16384
D = 64
N = 100000
NC = 2
NS = 16
NW = NC * NS
FPW = D // NW
CH = 4096
NCH = B // CH
HALF = B // 2
LANES = 16


@functools.partial(
    pl.kernel,
    out_type=jax.ShapeDtypeStruct((D, B), jnp.float32),
    mesh=plsc.VectorSubcoreMesh(core_axis_name="c", subcore_axis_name="s"),
    compiler_params=pltpu.CompilerParams(needs_layout_passes=False),
    scratch_types=[
        pltpu.VMEM((N,), jnp.float32),
        pltpu.VMEM((HALF,), jnp.float32),
        pltpu.VMEM((HALF,), jnp.float32),
        pltpu.VMEM((CH,), jnp.int32),
        pltpu.VMEM((CH,), jnp.int32),
        pltpu.SemaphoreType.DMA,
        pltpu.SemaphoreType.DMA,
        pltpu.SemaphoreType.DMA,
        pltpu.SemaphoreType.DMA,
        pltpu.SemaphoreType.DMA,
        pltpu.SemaphoreType.DMA,
    ],
)
def _plane_affine(xt_hbm, idx_hbm, st_hbm, ht_hbm, out_hbm,
                  plane_v, row0_v, row1_v, idx0_v, idx1_v,
                  sem_p, sem_x0, sem_x1, sem_i0, sem_i1, sem_o):
    wid = lax.axis_index("s") * NC + lax.axis_index("c")

    rows = (row0_v, row1_v)
    idxb = (idx0_v, idx1_v)
    isem = (sem_i0, sem_i1)
    xsem = (sem_x0, sem_x1)

    def fetch_idx(k):
        return pltpu.async_copy(
            idx_hbm.at[pl.ds(k * CH, CH)], idxb[k % 2], isem[k % 2])

    def sweep(k, mul):
        idx_ref = idxb[k % 2]
        row_ref = rows[k // 2]
        base = (k % 2) * CH

        @plsc.parallel_loop(0, CH, LANES, unroll=8)
        def body(i):
            iv = idx_ref[pl.ds(i, LANES)]
            g = plsc.load_gather(plane_v, [iv])
            s = pl.ds(base + i, LANES)
            if mul:
                row_ref[s] = row_ref[s] * g
            else:
                row_ref[s] = row_ref[s] + g

    idx_pref = fetch_idx(0)

    out_stores = []
    for f in range(FPW):
        j = wid * FPW + f

        with jax.named_scope("issue_splane"):
            cp_plane = pltpu.async_copy(st_hbm.at[j], plane_v, sem_p)
        with jax.named_scope("drain_out"):
            for cp in out_stores:
                cp.wait()
        out_stores = []
        with jax.named_scope("issue_x"):
            cp_x = [
                pltpu.async_copy(
                    xt_hbm.at[j, pl.ds(h * HALF, HALF)], rows[h], xsem[h])
                for h in range(2)
            ]
        with jax.named_scope("wait_splane"):
            cp_plane.wait()

        for k in range(NCH):
            with jax.named_scope("issue_idx"):
                nxt = fetch_idx(k + 1) if k + 1 < NCH else None
            with jax.named_scope("wait_x_idx"):
                if k % 2 == 0:
                    cp_x[k // 2].wait()
                idx_pref.wait()
            with jax.named_scope("sweep_mul"):
                sweep(k, mul=True)
            idx_pref = nxt

        with jax.named_scope("issue_hplane"):
            cp_plane = pltpu.async_copy(ht_hbm.at[j], plane_v, sem_p)
            idx_pref = fetch_idx(0)
        with jax.named_scope("wait_hplane"):
            cp_plane.wait()

        for k in range(NCH):
            with jax.named_scope("issue_idx"):
                nxt = fetch_idx(k + 1) if k + 1 < NCH else (
                    fetch_idx(0) if f + 1 < FPW else None)
            with jax.named_scope("wait_idx"):
                idx_pref.wait()
            with jax.named_scope("sweep_add"):
                sweep(k, mul=False)
            idx_pref = nxt
            if k % 2 == 1:
                h = k // 2
                with jax.named_scope("issue_out"):
                    out_stores.append(pltpu.async_copy(
                        rows[h], out_hbm.at[j, pl.ds(h * HALF, HALF)], sem_o))

    for cp in out_stores:
        cp.wait()


def kernel(x, batch_idx, scale_weight, shift_weight):
    idx = jnp.asarray(batch_idx, jnp.int32)
    out_t = _plane_affine(x.T, idx, scale_weight.T, shift_weight.T)
    return out_t.T


# speedup vs baseline: 1.0282x; 1.0282x over previous
import functools

import jax
import jax.numpy as jnp
from jax import lax
from jax.experimental import pallas as pl
from jax.experimental.pallas import tpu as pltpu
from jax.experimental.pallas import tpu_sc as plsc

B = 16384
D = 64
N = 100000
NC = 2
NS = 16
NW = NC * NS
FPW = D // NW
CH = 4096
NCH = B // CH
LANES = 16


@functools.partial(
    pl.kernel,
    out_type=jax.ShapeDtypeStruct((D, B), jnp.float32),
    mesh=plsc.VectorSubcoreMesh(core_axis_name="c", subcore_axis_name="s"),
    compiler_params=pltpu.CompilerParams(needs_layout_passes=False),
    scratch_types=[
        pltpu.VMEM((N,), jnp.float32),
        pltpu.VMEM((B,), jnp.float32),
        pltpu.VMEM((CH,), jnp.int32),
        pltpu.VMEM((CH,), jnp.int32),
        pltpu.SemaphoreType.DMA,
        pltpu.SemaphoreType.DMA,
        pltpu.SemaphoreType.DMA,
        pltpu.SemaphoreType.DMA,
        pltpu.SemaphoreType.DMA,
    ],
)
def _plane_affine(xt_hbm, idx_hbm, st_hbm, ht_hbm, out_hbm,
                  plane_v, row_v, idx0_v, idx1_v,
                  sem_p, sem_x, sem_i0, sem_i1, sem_o):
    wid = lax.axis_index("s") * NC + lax.axis_index("c")

    idxb = (idx0_v, idx1_v)
    isem = (sem_i0, sem_i1)

    def fetch_idx(c, buf):
        pltpu.async_copy(idx_hbm.at[pl.ds(c * CH, CH)], idxb[buf], isem[buf])

    def wait_idx(buf):
        pltpu.make_async_copy(
            idx_hbm.at[pl.ds(0, CH)], idxb[buf], isem[buf]).wait()

    def sweep(idx_ref, cbase, mul):
        @plsc.parallel_loop(0, CH, LANES, unroll=8)
        def body(i):
            iv = idx_ref[pl.ds(i, LANES)]
            g = plsc.load_gather(plane_v, [iv])
            s = pl.ds(cbase + i, LANES)
            if mul:
                row_v[s] = row_v[s] * g
            else:
                row_v[s] = row_v[s] + g

    def pass_(mul):
        def pair(p, carry):
            wait_idx(0)
            sweep(idx0_v, 2 * p * CH, mul)
            fetch_idx((2 * p + 2) % NCH, 0)
            wait_idx(1)
            sweep(idx1_v, (2 * p + 1) * CH, mul)
            fetch_idx((2 * p + 3) % NCH, 1)
            return carry

        lax.fori_loop(0, NCH // 2, pair, 0)

    fetch_idx(0, 0)
    fetch_idx(1, 1)

    def feat(f, carry):
        j = wid * FPW + f
        pltpu.async_copy(xt_hbm.at[j], row_v, sem_x)
        pltpu.async_copy(st_hbm.at[j], plane_v, sem_p)
        pltpu.make_async_copy(st_hbm.at[j], plane_v, sem_p).wait()
        pltpu.make_async_copy(xt_hbm.at[j], row_v, sem_x).wait()
        pass_(mul=True)
        pltpu.async_copy(ht_hbm.at[j], plane_v, sem_p)
        pltpu.make_async_copy(ht_hbm.at[j], plane_v, sem_p).wait()
        pass_(mul=False)
        pltpu.async_copy(row_v, out_hbm.at[j], sem_o)
        pltpu.make_async_copy(row_v, out_hbm.at[j], sem_o).wait()
        return carry

    lax.fori_loop(0, FPW, feat, 0)

    wait_idx(0)
    wait_idx(1)


def kernel(x, batch_idx, scale_weight, shift_weight):
    idx = jnp.asarray(batch_idx, jnp.int32)
    out_t = _plane_affine(x.T, idx, scale_weight.T, shift_weight.T)
    return out_t.T


# speedup vs baseline: 1.0431x; 1.0145x over previous
import functools

import jax
import jax.numpy as jnp
from jax import lax
from jax.experimental import pallas as pl
from jax.experimental.pallas import tpu as pltpu
from jax.experimental.pallas import tpu_sc as plsc

B = 16384
D = 64
N = 100000
NC = 2
NS = 16
NW = NC * NS
FPW = D // NW
CH = 2048
NCH = B // CH
LANES = 16


@functools.partial(
    pl.kernel,
    out_type=jax.ShapeDtypeStruct((D, B), jnp.float32),
    mesh=plsc.VectorSubcoreMesh(core_axis_name="c", subcore_axis_name="s"),
    compiler_params=pltpu.CompilerParams(needs_layout_passes=False),
    scratch_types=[
        pltpu.VMEM((N,), jnp.float32),
        pltpu.VMEM((B,), jnp.float32),
        pltpu.VMEM((CH,), jnp.int32),
        pltpu.VMEM((CH,), jnp.int32),
        pltpu.VMEM((CH,), jnp.int32),
        pltpu.VMEM((CH,), jnp.int32),
        pltpu.SemaphoreType.DMA,
        pltpu.SemaphoreType.DMA,
        pltpu.SemaphoreType.DMA,
        pltpu.SemaphoreType.DMA,
        pltpu.SemaphoreType.DMA,
        pltpu.SemaphoreType.DMA,
        pltpu.SemaphoreType.DMA,
    ],
)
def _plane_affine(xt_hbm, idx_hbm, st_hbm, ht_hbm, out_hbm,
                  plane_v, row_v, idx0_v, idx1_v, idx2_v, idx3_v,
                  sem_p, sem_x, sem_i0, sem_i1, sem_i2, sem_i3, sem_o):
    wid = lax.axis_index("s") * NC + lax.axis_index("c")

    idxb = (idx0_v, idx1_v, idx2_v, idx3_v)
    isem = (sem_i0, sem_i1, sem_i2, sem_i3)

    def fetch_idx(c, buf):
        pltpu.async_copy(idx_hbm.at[pl.ds(c * CH, CH)], idxb[buf], isem[buf])

    def wait_idx(buf):
        pltpu.make_async_copy(
            idx_hbm.at[pl.ds(0, CH)], idxb[buf], isem[buf]).wait()

    def sweep(idx_ref, cbase, mul):
        @plsc.parallel_loop(0, CH, LANES, unroll=8)
        def body(i):
            iv = idx_ref[pl.ds(i, LANES)]
            g = plsc.load_gather(plane_v, [iv])
            s = pl.ds(cbase + i, LANES)
            if mul:
                row_v[s] = row_v[s] * g
            else:
                row_v[s] = row_v[s] + g

    def pass_(mul):
        def quad(q, carry):
            for u in range(4):
                c = 4 * q + u
                wait_idx(u)
                sweep(idxb[u], c * CH, mul)
                fetch_idx((c + 4) % NCH, u)
            return carry

        lax.fori_loop(0, NCH // 4, quad, 0)

    for u in range(4):
        fetch_idx(u, u)

    def feat(f, carry):
        j = wid * FPW + f
        pltpu.async_copy(xt_hbm.at[j], row_v, sem_x)
        pltpu.async_copy(st_hbm.at[j], plane_v, sem_p)
        pltpu.make_async_copy(st_hbm.at[j], plane_v, sem_p).wait()
        pltpu.make_async_copy(xt_hbm.at[j], row_v, sem_x).wait()
        pass_(mul=True)
        pltpu.async_copy(ht_hbm.at[j], plane_v, sem_p)
        pltpu.make_async_copy(ht_hbm.at[j], plane_v, sem_p).wait()
        pass_(mul=False)
        pltpu.async_copy(row_v, out_hbm.at[j], sem_o)
        pltpu.make_async_copy(row_v, out_hbm.at[j], sem_o).wait()
        return carry

    lax.fori_loop(0, FPW, feat, 0)

    for u in range(4):
        wait_idx(u)


def kernel(x, batch_idx, scale_weight, shift_weight):
    idx = jnp.asarray(batch_idx, jnp.int32)
    out_t = _plane_affine(x.T, idx, scale_weight.T, shift_weight.T)
    return out_t.T
